# Initial kernel scaffold; baseline (speedup 1.0000x reference)
import jax
import jax.numpy as jnp
from jax.experimental import pallas as pl


def kernel(x, edge_index_0, edge_index_1, edge_index_2, W_0, W_1, W_2, W_root, b_root):
    raise NotImplementedError("write your pallas kernel here")


# speedup vs baseline: 1.2317x; 1.2317x over previous
import functools

import jax
import jax.numpy as jnp
from jax import lax
from jax.experimental import pallas as pl
from jax.experimental.pallas import tpu as pltpu
from jax.experimental.pallas import tpu_sc as plsc

_NT = 16
_CH = 128
_DUM = 64
_RC = 32


def _mm_body(x_ref, w_ref, b_ref, msgs_ref, root_ref):
    h = jnp.dot(x_ref[...], w_ref[...], preferred_element_type=jnp.float32)
    msgs_ref[...] = h[:, 128:]
    root_ref[...] = h[:, :128] + b_ref[...]


@functools.lru_cache(maxsize=None)
def _build_tc(NP, D, bn):
    return pl.pallas_call(
        _mm_body,
        grid=(NP // bn,),
        in_specs=[
            pl.BlockSpec((bn, D), lambda i: (i, 0)),
            pl.BlockSpec((D, 4 * D), lambda i: (0, 0)),
            pl.BlockSpec((1, D), lambda i: (0, 0)),
        ],
        out_specs=[
            pl.BlockSpec((bn, 3 * D), lambda i: (i, 0)),
            pl.BlockSpec((bn, D), lambda i: (i, 0)),
        ],
        out_shape=[
            jax.ShapeDtypeStruct((NP, 3 * D), jnp.float32),
            jax.ShapeDtypeStruct((NP, D), jnp.float32),
        ],
    )


@functools.lru_cache(maxsize=None)
def _build_sc(NP, nch):
    rng = NP // 4
    n_rchunks = rng // _RC
    rounds = -(-n_rchunks // _NT)
    crows = _CH // 128
    erows_per_tile = nch * crows
    erows_per_rel = erows_per_tile * _NT
    mesh = plsc.VectorSubcoreMesh(core_axis_name="c", subcore_axis_name="s")

    @functools.partial(
        pl.kernel,
        mesh=mesh,
        out_type=jax.ShapeDtypeStruct((NP, 128), jnp.float32),
        scratch_types=[
            pltpu.VMEM((crows, 128), jnp.int32),
            pltpu.VMEM((crows, 128), jnp.int32),
            pltpu.VMEM((_CH, 128), jnp.float32),
            pltpu.VMEM_SHARED((rng + _DUM, 128), jnp.float32),
            pltpu.SemaphoreType.DMA,
        ],
    )
    def sc_fn(msgs_hbm, root_hbm, edges_hbm, out_hbm,
              src_v, dst_v, msg_v, acc, sem):
        c = lax.axis_index("c")
        s = lax.axis_index("s")
        for qq in range(2):
            q = c * 2 + qq
            base_node = q * rng
            for m in range(rounds):
                cid = s + m * _NT

                @pl.when(cid < n_rchunks)
                def _():
                    pltpu.sync_copy(
                        root_hbm.at[pl.ds(base_node + cid * _RC, _RC), :],
                        acc.at[pl.ds(cid * _RC, _RC), :])
            plsc.subcore_barrier()
            for r in range(3):
                ebase = r * erows_per_rel + s * erows_per_tile

                def chunk(j, carry):
                    ro = ebase + j * crows
                    pltpu.sync_copy(edges_hbm.at[0, pl.ds(ro, crows), :], src_v)
                    pltpu.sync_copy(edges_hbm.at[1, pl.ds(ro, crows), :], dst_v)
                    for i in range(crows):
                        for k in range(8):
                            sl = pl.ds(k * 16, 16)
                            sv = src_v[i, sl]
                            dv = dst_v[i, sl]
                            src_v[i, sl] = sv * 3 + r
                            loc = dv - base_node
                            ok = (loc >= 0) & (loc < rng)
                            dum = rng + (dv & (_DUM - 1))
                            dst_v[i, sl] = jnp.where(ok, loc, dum)
                    handles = []
                    for i in range(crows):
                        handles.append(pltpu.async_copy(
                            msgs_hbm.at[src_v.at[i]],
                            msg_v.at[pl.ds(i * 128, 128), :], sem))
                    for h in handles:
                        h.wait()
                    for i in range(crows):
                        pltpu.sync_copy(msg_v.at[pl.ds(i * 128, 128), :],
                                        acc.at[dst_v.at[i]], add=True)
                    return carry

                lax.fori_loop(0, nch, chunk, 0)
            plsc.subcore_barrier()
            for m in range(rounds):
                cid = s + m * _NT

                @pl.when(cid < n_rchunks)
                def _():
                    pltpu.sync_copy(
                        acc.at[pl.ds(cid * _RC, _RC), :],
                        out_hbm.at[pl.ds(base_node + cid * _RC, _RC), :])

    return sc_fn


def kernel(x, edge_index_0, edge_index_1, edge_index_2, W_0, W_1, W_2, W_root, b_root):
    N, D = x.shape
    E = edge_index_0.shape[1]
    NP = -(-N // 128) * 128

    w_all = jnp.concatenate([W_root.T, W_0.T, W_1.T, W_2.T], axis=1)
    xp = jnp.concatenate([x, jnp.zeros((NP - N, D), x.dtype)])
    msgs, root = _build_tc(NP, D, 3128)(xp, w_all, b_root.reshape(1, D))

    nch = -(-E // (_CH * _NT))
    epr = nch * _CH * _NT
    srcs, dsts = [], []
    for e in (edge_index_0, edge_index_1, edge_index_2):
        srcs.append(jnp.concatenate([e[0], jnp.zeros((epr - E,), jnp.int32)]))
        dsts.append(jnp.concatenate([e[1], jnp.full((epr - E,), N, jnp.int32)]))
    edges = jnp.stack([jnp.concatenate(srcs), jnp.concatenate(dsts)])
    edges = edges.reshape(2, 3 * epr // 128, 128)

    out = _build_sc(NP, nch)(msgs.reshape(NP * 3, D), root, edges)
    return out[:N]


# speedup vs baseline: 1.8142x; 1.4729x over previous
import functools

import jax
import jax.numpy as jnp
from jax import lax
from jax.experimental import pallas as pl
from jax.experimental.pallas import tpu as pltpu
from jax.experimental.pallas import tpu_sc as plsc

_NT = 16
_CH = 128
_DUM = 64
_RC = 32


def _mm_body(x_ref, w_ref, b_ref, msgs_ref, root_ref):
    h = jnp.dot(x_ref[...], w_ref[...], preferred_element_type=jnp.float32)
    msgs_ref[...] = h[:, 128:]
    root_ref[...] = h[:, :128] + b_ref[...]


@functools.lru_cache(maxsize=None)
def _build_tc(NP, D, bn):
    return pl.pallas_call(
        _mm_body,
        grid=(NP // bn,),
        in_specs=[
            pl.BlockSpec((bn, D), lambda i: (i, 0)),
            pl.BlockSpec((D, 4 * D), lambda i: (0, 0)),
            pl.BlockSpec((1, D), lambda i: (0, 0)),
        ],
        out_specs=[
            pl.BlockSpec((bn, 3 * D), lambda i: (i, 0)),
            pl.BlockSpec((bn, D), lambda i: (i, 0)),
        ],
        out_shape=[
            jax.ShapeDtypeStruct((NP, 3 * D), jnp.float32),
            jax.ShapeDtypeStruct((NP, D), jnp.float32),
        ],
    )


@functools.lru_cache(maxsize=None)
def _build_sc(NP, nch):
    rng = NP // 4
    n_rchunks = rng // _RC
    rounds = -(-n_rchunks // _NT)
    crows = _CH // 128
    erows_per_tile = nch * crows
    erows_per_rel = erows_per_tile * _NT
    mesh = plsc.VectorSubcoreMesh(core_axis_name="c", subcore_axis_name="s")

    @functools.partial(
        pl.kernel,
        mesh=mesh,
        out_type=jax.ShapeDtypeStruct((NP, 128), jnp.float32),
        scratch_types=[
            pltpu.VMEM((2, 2, 1, 128), jnp.int32),
            pltpu.VMEM((2, 4, 32), jnp.int32),
            pltpu.VMEM((2, 4, 32), jnp.int32),
            pltpu.VMEM((4, 32, 128), jnp.float32),
            pltpu.VMEM_SHARED((rng + _DUM, 128), jnp.float32),
            pltpu.SemaphoreType.DMA,
            pltpu.SemaphoreType.DMA,
            pltpu.SemaphoreType.DMA,
        ],
    )
    def sc_fn(msgs_hbm, root_hbm, edges_hbm, out_hbm,
              ed_v, gidx_v, ldst_v, msg_v, acc, sem_g, sem_s, sem_e):
        c = lax.axis_index("c")
        s = lax.axis_index("s")
        for qq in range(2):
            q = c * 2 + qq
            base_node = q * rng
            for m in range(rounds):
                cid = s + m * _NT

                @pl.when(cid < n_rchunks)
                def _():
                    pltpu.sync_copy(
                        root_hbm.at[pl.ds(base_node + cid * _RC, _RC), :],
                        acc.at[pl.ds(cid * _RC, _RC), :])
            plsc.subcore_barrier()
            for r in range(3):
                ebase = r * erows_per_rel + s * erows_per_tile
                pltpu.async_copy(edges_hbm.at[:, pl.ds(ebase, crows), :],
                                 ed_v.at[0], sem_e)

                def chunk(j, carry):
                    pj = lax.rem(j, 2)
                    pltpu.make_async_copy(edges_hbm.at[:, pl.ds(0, crows), :],
                                          ed_v.at[0], sem_e).wait()

                    @pl.when(j + 1 < nch)
                    def _():
                        ro = ebase + (j + 1) * crows
                        pltpu.async_copy(edges_hbm.at[:, pl.ds(ro, crows), :],
                                         ed_v.at[1 - pj], sem_e)
                    for b in range(4):
                        for k in range(2):
                            sl = pl.ds(k * 16, 16)
                            el = pl.ds(b * 32 + k * 16, 16)
                            sv = ed_v[pj, 0, 0, el]
                            dv = ed_v[pj, 1, 0, el]
                            gidx_v[pj, b, sl] = sv * 3 + r
                            loc = dv - base_node
                            ok = (loc >= 0) & (loc < rng)
                            dum = rng + (dv & (_DUM - 1))
                            ldst_v[pj, b, sl] = jnp.where(ok, loc, dum)
                    handles = []
                    for b in range(4):
                        @pl.when(j > 0)
                        def _():
                            pltpu.make_async_copy(
                                msgs_hbm.at[pl.ds(0, 32), :],
                                msg_v.at[b], sem_s).wait()
                        handles.append(pltpu.async_copy(
                            msgs_hbm.at[gidx_v.at[pj, b]],
                            msg_v.at[b], sem_g))
                    for b in range(4):
                        handles[b].wait()
                        pltpu.async_copy(msg_v.at[b],
                                         acc.at[ldst_v.at[pj, b]],
                                         sem_s, add=True)
                    return carry

                lax.fori_loop(0, nch, chunk, 0)
                for b in range(4):
                    pltpu.make_async_copy(
                        msgs_hbm.at[pl.ds(0, 32), :],
                        msg_v.at[b], sem_s).wait()
            plsc.subcore_barrier()
            for m in range(rounds):
                cid = s + m * _NT

                @pl.when(cid < n_rchunks)
                def _():
                    pltpu.sync_copy(
                        acc.at[pl.ds(cid * _RC, _RC), :],
                        out_hbm.at[pl.ds(base_node + cid * _RC, _RC), :])

    return sc_fn


def kernel(x, edge_index_0, edge_index_1, edge_index_2, W_0, W_1, W_2, W_root, b_root):
    N, D = x.shape
    E = edge_index_0.shape[1]
    NP = -(-N // 128) * 128

    w_all = jnp.concatenate([W_root.T, W_0.T, W_1.T, W_2.T], axis=1)
    xp = jnp.concatenate([x, jnp.zeros((NP - N, D), x.dtype)])
    msgs, root = _build_tc(NP, D, 3128)(xp, w_all, b_root.reshape(1, D))

    nch = -(-E // (_CH * _NT))
    epr = nch * _CH * _NT
    srcs, dsts = [], []
    for e in (edge_index_0, edge_index_1, edge_index_2):
        srcs.append(jnp.concatenate([e[0], jnp.zeros((epr - E,), jnp.int32)]))
        dsts.append(jnp.concatenate([e[1], jnp.full((epr - E,), N, jnp.int32)]))
    edges = jnp.stack([jnp.concatenate(srcs), jnp.concatenate(dsts)])
    edges = edges.reshape(2, 3 * epr // 128, 128)

    out = _build_sc(NP, nch)(msgs.reshape(NP * 3, D), root, edges)
    return out[:N]


# speedup vs baseline: 2.7112x; 1.4944x over previous
import functools

import jax
import jax.numpy as jnp
from jax import lax
from jax.experimental import pallas as pl
from jax.experimental.pallas import tpu as pltpu
from jax.experimental.pallas import tpu_sc as plsc

_NT = 16
_CH = 128
_RC = 32


def _mm_body(x_ref, w_ref, b_ref, msgs_ref, root_ref):
    h = jnp.dot(x_ref[...], w_ref[...], preferred_element_type=jnp.float32)
    msgs_ref[...] = h[:, 128:]
    root_ref[...] = h[:, :128] + b_ref[...]


@functools.lru_cache(maxsize=None)
def _build_tc(NP, D, bn):
    return pl.pallas_call(
        _mm_body,
        grid=(NP // bn,),
        in_specs=[
            pl.BlockSpec((bn, D), lambda i: (i, 0)),
            pl.BlockSpec((D, 4 * D), lambda i: (0, 0)),
            pl.BlockSpec((1, D), lambda i: (0, 0)),
        ],
        out_specs=[
            pl.BlockSpec((bn, 3 * D), lambda i: (i, 0)),
            pl.BlockSpec((bn, D), lambda i: (i, 0)),
        ],
        out_shape=[
            jax.ShapeDtypeStruct((NP, 3 * D), jnp.float32),
            jax.ShapeDtypeStruct((NP, D), jnp.float32),
        ],
    )


@functools.lru_cache(maxsize=None)
def _build_sc(NP, nch):
    n_rchunks = NP // _RC
    rounds = -(-n_rchunks // _NT)
    crows = _CH // 128
    erows_per_tile = nch * crows
    erows_per_rel = erows_per_tile * _NT
    mesh = plsc.VectorSubcoreMesh(core_axis_name="c", subcore_axis_name="s")

    @functools.partial(
        pl.kernel,
        mesh=mesh,
        out_type=jax.ShapeDtypeStruct((NP, 128), jnp.float32),
        compiler_params=pltpu.CompilerParams(use_tc_tiling_on_sc=False),
        scratch_types=[
            pltpu.VMEM((2, 2, 1, 128), jnp.int32),
            pltpu.VMEM((2, 4, 32), jnp.int32),
            pltpu.VMEM((2, 4, 32), jnp.int32),
            pltpu.VMEM((4, 32, 32), jnp.float32),
            pltpu.VMEM_SHARED((NP + 8, 32), jnp.float32),
            pltpu.SemaphoreType.DMA,
            pltpu.SemaphoreType.DMA,
            pltpu.SemaphoreType.DMA,
        ],
    )
    def sc_fn(msgs_hbm, root_hbm, edges_hbm, out_hbm,
              ed_v, gidx_v, ldst_v, msg_v, acc, sem_g, sem_s, sem_e):
        c = lax.axis_index("c")
        s = lax.axis_index("s")
        for gg in range(2):
            g = c * 2 + gg
            for m in range(rounds):
                cid = s + m * _NT

                @pl.when(cid < n_rchunks)
                def _():
                    pltpu.sync_copy(
                        root_hbm.at[pl.ds(cid * _RC, _RC), pl.ds(g * 32, 32)],
                        acc.at[pl.ds(cid * _RC, _RC), :])
            plsc.subcore_barrier()
            for r in range(3):
                ebase = r * erows_per_rel + s * erows_per_tile
                rg = r * 4 + g
                pltpu.async_copy(edges_hbm.at[:, pl.ds(ebase, crows), :],
                                 ed_v.at[0], sem_e)

                def chunk(j, carry):
                    pj = lax.rem(j, 2)
                    pltpu.make_async_copy(edges_hbm.at[:, pl.ds(0, crows), :],
                                          ed_v.at[0], sem_e).wait()

                    @pl.when(j + 1 < nch)
                    def _():
                        ro = ebase + (j + 1) * crows
                        pltpu.async_copy(edges_hbm.at[:, pl.ds(ro, crows), :],
                                         ed_v.at[1 - pj], sem_e)
                    for b in range(4):
                        for k in range(2):
                            sl = pl.ds(k * 16, 16)
                            el = pl.ds(b * 32 + k * 16, 16)
                            gidx_v[pj, b, sl] = ed_v[pj, 0, 0, el] * 12 + rg
                            ldst_v[pj, b, sl] = ed_v[pj, 1, 0, el]
                    handles = []
                    for b in range(4):
                        @pl.when(j > 0)
                        def _():
                            pltpu.make_async_copy(
                                msgs_hbm.at[pl.ds(0, 32), :],
                                msg_v.at[b], sem_s).wait()
                        handles.append(pltpu.async_copy(
                            msgs_hbm.at[gidx_v.at[pj, b]],
                            msg_v.at[b], sem_g))
                    for b in range(4):
                        handles[b].wait()
                        pltpu.async_copy(msg_v.at[b],
                                         acc.at[ldst_v.at[pj, b]],
                                         sem_s, add=True)
                    return carry

                lax.fori_loop(0, nch, chunk, 0)
                for b in range(4):
                    pltpu.make_async_copy(
                        msgs_hbm.at[pl.ds(0, 32), :],
                        msg_v.at[b], sem_s).wait()
            plsc.subcore_barrier()
            for m in range(rounds):
                cid = s + m * _NT

                @pl.when(cid < n_rchunks)
                def _():
                    pltpu.sync_copy(
                        acc.at[pl.ds(cid * _RC, _RC), :],
                        out_hbm.at[pl.ds(cid * _RC, _RC), pl.ds(g * 32, 32)])

    return sc_fn


def kernel(x, edge_index_0, edge_index_1, edge_index_2, W_0, W_1, W_2, W_root, b_root):
    N, D = x.shape
    E = edge_index_0.shape[1]
    NP = -(-N // 128) * 128

    w_all = jnp.concatenate([W_root.T, W_0.T, W_1.T, W_2.T], axis=1)
    xp = jnp.concatenate([x, jnp.zeros((NP - N, D), x.dtype)])
    msgs, root = _build_tc(NP, D, 3128)(xp, w_all, b_root.reshape(1, D))

    nch = -(-E // (_CH * _NT))
    epr = nch * _CH * _NT
    srcs, dsts = [], []
    for e in (edge_index_0, edge_index_1, edge_index_2):
        srcs.append(jnp.concatenate([e[0], jnp.zeros((epr - E,), jnp.int32)]))
        dsts.append(jnp.concatenate([e[1], jnp.full((epr - E,), N, jnp.int32)]))
    edges = jnp.stack([jnp.concatenate(srcs), jnp.concatenate(dsts)])
    edges = edges.reshape(2, 3 * epr // 128, 128)

    out = _build_sc(NP, nch)(msgs.reshape(NP * 12, 32), root, edges)
    return out[:N]


# speedup vs baseline: 3.2016x; 1.1809x over previous
import functools

import jax
import jax.numpy as jnp
from jax import lax
from jax.experimental import pallas as pl
from jax.experimental.pallas import tpu as pltpu
from jax.experimental.pallas import tpu_sc as plsc

_NT = 16
_CH = 256
_QE = _CH // 4
_RC = 32


def _mm_body(x_ref, w_ref, b_ref, msgs_ref, root_ref):
    h = jnp.dot(x_ref[...], w_ref[...], preferred_element_type=jnp.float32)
    msgs_ref[...] = h[:, 128:]
    root_ref[...] = h[:, :128] + b_ref[...]


@functools.lru_cache(maxsize=None)
def _build_tc(NP, D, bn):
    return pl.pallas_call(
        _mm_body,
        grid=(NP // bn,),
        in_specs=[
            pl.BlockSpec((bn, D), lambda i: (i, 0)),
            pl.BlockSpec((D, 4 * D), lambda i: (0, 0)),
            pl.BlockSpec((1, D), lambda i: (0, 0)),
        ],
        out_specs=[
            pl.BlockSpec((bn, 3 * D), lambda i: (i, 0)),
            pl.BlockSpec((bn, D), lambda i: (i, 0)),
        ],
        out_shape=[
            jax.ShapeDtypeStruct((NP, 3 * D), jnp.float32),
            jax.ShapeDtypeStruct((NP, D), jnp.float32),
        ],
    )


@functools.lru_cache(maxsize=None)
def _build_sc(NP, nch):
    n_rchunks = NP // _RC
    rounds = -(-n_rchunks // _NT)
    crows = _CH // 128
    erows_per_tile = nch * crows
    erows_per_rel = erows_per_tile * _NT
    mesh = plsc.VectorSubcoreMesh(core_axis_name="c", subcore_axis_name="s")

    @functools.partial(
        pl.kernel,
        mesh=mesh,
        out_type=jax.ShapeDtypeStruct((NP, 128), jnp.float32),
        compiler_params=pltpu.CompilerParams(use_tc_tiling_on_sc=False),
        scratch_types=[
            pltpu.VMEM((2, 2, _CH // 128, 128), jnp.int32),
            pltpu.VMEM((2, 4, _QE), jnp.int32),
            pltpu.VMEM((2, 4, _QE), jnp.int32),
            pltpu.VMEM((4, _QE, 32), jnp.float32),
            pltpu.VMEM_SHARED((NP + 8, 32), jnp.float32),
            pltpu.SemaphoreType.DMA,
            pltpu.SemaphoreType.DMA,
            pltpu.SemaphoreType.DMA,
        ],
    )
    def sc_fn(msgs_hbm, root_hbm, edges_hbm, out_hbm,
              ed_v, gidx_v, ldst_v, msg_v, acc, sem_g, sem_s, sem_e):
        c = lax.axis_index("c")
        s = lax.axis_index("s")
        for gg in range(2):
            g = c * 2 + gg
            for m in range(rounds):
                cid = s + m * _NT

                @pl.when(cid < n_rchunks)
                def _():
                    pltpu.sync_copy(
                        root_hbm.at[pl.ds(cid * _RC, _RC), pl.ds(g * 32, 32)],
                        acc.at[pl.ds(cid * _RC, _RC), :])
            plsc.subcore_barrier()
            for r in range(3):
                ebase = r * erows_per_rel + s * erows_per_tile
                rg = r * 4 + g
                pltpu.async_copy(edges_hbm.at[:, pl.ds(ebase, crows), :],
                                 ed_v.at[0], sem_e)

                def chunk(j, carry):
                    pj = lax.rem(j, 2)
                    pltpu.make_async_copy(edges_hbm.at[:, pl.ds(0, crows), :],
                                          ed_v.at[0], sem_e).wait()

                    @pl.when(j + 1 < nch)
                    def _():
                        ro = ebase + (j + 1) * crows
                        pltpu.async_copy(edges_hbm.at[:, pl.ds(ro, crows), :],
                                         ed_v.at[1 - pj], sem_e)
                    for b in range(4):
                        for k in range(_QE // 16):
                            pos = b * _QE + k * 16
                            sl = pl.ds(k * 16, 16)
                            el = pl.ds(pos % 128, 16)
                            gidx_v[pj, b, sl] = ed_v[pj, 0, pos // 128, el] * 12 + rg
                            ldst_v[pj, b, sl] = ed_v[pj, 1, pos // 128, el]
                    handles = []
                    for b in range(4):
                        @pl.when(j > 0)
                        def _():
                            pltpu.make_async_copy(
                                msgs_hbm.at[pl.ds(0, _QE), :],
                                msg_v.at[b], sem_s).wait()
                        handles.append(pltpu.async_copy(
                            msgs_hbm.at[gidx_v.at[pj, b]],
                            msg_v.at[b], sem_g))
                    for b in range(4):
                        handles[b].wait()
                        pltpu.async_copy(msg_v.at[b],
                                         acc.at[ldst_v.at[pj, b]],
                                         sem_s, add=True)
                    return carry

                lax.fori_loop(0, nch, chunk, 0)
                for b in range(4):
                    pltpu.make_async_copy(
                        msgs_hbm.at[pl.ds(0, _QE), :],
                        msg_v.at[b], sem_s).wait()
            plsc.subcore_barrier()
            for m in range(rounds):
                cid = s + m * _NT

                @pl.when(cid < n_rchunks)
                def _():
                    pltpu.sync_copy(
                        acc.at[pl.ds(cid * _RC, _RC), :],
                        out_hbm.at[pl.ds(cid * _RC, _RC), pl.ds(g * 32, 32)])

    return sc_fn


def kernel(x, edge_index_0, edge_index_1, edge_index_2, W_0, W_1, W_2, W_root, b_root):
    N, D = x.shape
    E = edge_index_0.shape[1]
    NP = -(-N // 128) * 128

    w_all = jnp.concatenate([W_root.T, W_0.T, W_1.T, W_2.T], axis=1)
    xp = jnp.concatenate([x, jnp.zeros((NP - N, D), x.dtype)])
    msgs, root = _build_tc(NP, D, 3128)(xp, w_all, b_root.reshape(1, D))

    nch = -(-E // (_CH * _NT))
    epr = nch * _CH * _NT
    srcs, dsts = [], []
    for e in (edge_index_0, edge_index_1, edge_index_2):
        srcs.append(jnp.concatenate([e[0], jnp.zeros((epr - E,), jnp.int32)]))
        dsts.append(jnp.concatenate([e[1], jnp.full((epr - E,), N, jnp.int32)]))
    edges = jnp.stack([jnp.concatenate(srcs), jnp.concatenate(dsts)])
    edges = edges.reshape(2, 3 * epr // 128, 128)

    out = _build_sc(NP, nch)(msgs.reshape(NP * 12, 32), root, edges)
    return out[:N]


# speedup vs baseline: 4.2036x; 1.3129x over previous
import functools

import jax
import jax.numpy as jnp
from jax import lax
from jax.experimental import pallas as pl
from jax.experimental.pallas import tpu as pltpu
from jax.experimental.pallas import tpu_sc as plsc

_NT = 16
_CH = 256
_QE = _CH // 4
_RC = 32


def _mm_body(x_ref, w_ref, b_ref, msgs_ref, root_ref):
    h = jnp.dot(x_ref[...], w_ref[...], preferred_element_type=jnp.float32)
    msgs_ref[...] = h[:, 128:].astype(jnp.bfloat16)
    root_ref[...] = (h[:, :128] + b_ref[...]).astype(jnp.bfloat16)


@functools.lru_cache(maxsize=None)
def _build_tc(NP, D, bn):
    return pl.pallas_call(
        _mm_body,
        grid=(NP // bn,),
        in_specs=[
            pl.BlockSpec((bn, D), lambda i: (i, 0)),
            pl.BlockSpec((D, 4 * D), lambda i: (0, 0)),
            pl.BlockSpec((1, D), lambda i: (0, 0)),
        ],
        out_specs=[
            pl.BlockSpec((bn, 3 * D), lambda i: (i, 0)),
            pl.BlockSpec((bn, D), lambda i: (i, 0)),
        ],
        out_shape=[
            jax.ShapeDtypeStruct((NP, 3 * D), jnp.bfloat16),
            jax.ShapeDtypeStruct((NP, D), jnp.bfloat16),
        ],
    )


@functools.lru_cache(maxsize=None)
def _build_sc(NP, nch):
    n_rchunks = NP // _RC
    rounds = -(-n_rchunks // _NT)
    crows = _CH // 128
    erows_per_tile = nch * crows
    erows_per_rel = erows_per_tile * _NT
    mesh = plsc.VectorSubcoreMesh(core_axis_name="c", subcore_axis_name="s")

    @functools.partial(
        pl.kernel,
        mesh=mesh,
        out_type=jax.ShapeDtypeStruct((NP, 128), jnp.bfloat16),
        compiler_params=pltpu.CompilerParams(use_tc_tiling_on_sc=False),
        scratch_types=[
            pltpu.VMEM((2, 2, _CH // 128, 128), jnp.int32),
            pltpu.VMEM((2, 4, _QE), jnp.int32),
            pltpu.VMEM((2, 4, _QE), jnp.int32),
            pltpu.VMEM((4, _QE, 64), jnp.bfloat16),
            pltpu.VMEM_SHARED((NP + 8, 64), jnp.bfloat16),
            pltpu.SemaphoreType.DMA,
            pltpu.SemaphoreType.DMA,
            pltpu.SemaphoreType.DMA,
        ],
    )
    def sc_fn(msgs_hbm, root_hbm, edges_hbm, out_hbm,
              ed_v, gidx_v, ldst_v, msg_v, acc, sem_g, sem_s, sem_e):
        c = lax.axis_index("c")
        s = lax.axis_index("s")
        for gg in range(1):
            g = c
            for m in range(rounds):
                cid = s + m * _NT

                @pl.when(cid < n_rchunks)
                def _():
                    pltpu.sync_copy(
                        root_hbm.at[pl.ds(cid * _RC, _RC), pl.ds(g * 64, 64)],
                        acc.at[pl.ds(cid * _RC, _RC), :])
            plsc.subcore_barrier()
            for r in range(3):
                ebase = r * erows_per_rel + s * erows_per_tile
                rg = r * 2 + g
                pltpu.async_copy(edges_hbm.at[:, pl.ds(ebase, crows), :],
                                 ed_v.at[0], sem_e)

                def chunk(j, carry):
                    pj = lax.rem(j, 2)
                    pltpu.make_async_copy(edges_hbm.at[:, pl.ds(0, crows), :],
                                          ed_v.at[0], sem_e).wait()

                    @pl.when(j + 1 < nch)
                    def _():
                        ro = ebase + (j + 1) * crows
                        pltpu.async_copy(edges_hbm.at[:, pl.ds(ro, crows), :],
                                         ed_v.at[1 - pj], sem_e)
                    for b in range(4):
                        for k in range(_QE // 16):
                            pos = b * _QE + k * 16
                            sl = pl.ds(k * 16, 16)
                            el = pl.ds(pos % 128, 16)
                            gidx_v[pj, b, sl] = ed_v[pj, 0, pos // 128, el] * 6 + rg
                            ldst_v[pj, b, sl] = ed_v[pj, 1, pos // 128, el]
                    handles = []
                    for b in range(4):
                        @pl.when(j > 0)
                        def _():
                            pltpu.make_async_copy(
                                msgs_hbm.at[pl.ds(0, _QE), :],
                                msg_v.at[b], sem_s).wait()
                        handles.append(pltpu.async_copy(
                            msgs_hbm.at[gidx_v.at[pj, b]],
                            msg_v.at[b], sem_g))
                    for b in range(4):
                        handles[b].wait()
                        pltpu.async_copy(msg_v.at[b],
                                         acc.at[ldst_v.at[pj, b]],
                                         sem_s, add=True)
                    return carry

                lax.fori_loop(0, nch, chunk, 0)
                for b in range(4):
                    pltpu.make_async_copy(
                        msgs_hbm.at[pl.ds(0, _QE), :],
                        msg_v.at[b], sem_s).wait()
            plsc.subcore_barrier()
            for m in range(rounds):
                cid = s + m * _NT

                @pl.when(cid < n_rchunks)
                def _():
                    pltpu.sync_copy(
                        acc.at[pl.ds(cid * _RC, _RC), :],
                        out_hbm.at[pl.ds(cid * _RC, _RC), pl.ds(g * 64, 64)])

    return sc_fn


def kernel(x, edge_index_0, edge_index_1, edge_index_2, W_0, W_1, W_2, W_root, b_root):
    N, D = x.shape
    E = edge_index_0.shape[1]
    NP = -(-N // 128) * 128

    w_all = jnp.concatenate([W_root.T, W_0.T, W_1.T, W_2.T], axis=1)
    xp = jnp.concatenate([x, jnp.zeros((NP - N, D), x.dtype)])
    msgs, root = _build_tc(NP, D, 2176)(xp, w_all, b_root.reshape(1, D))

    nch = -(-E // (_CH * _NT))
    epr = nch * _CH * _NT
    srcs, dsts = [], []
    for e in (edge_index_0, edge_index_1, edge_index_2):
        srcs.append(jnp.concatenate([e[0], jnp.zeros((epr - E,), jnp.int32)]))
        dsts.append(jnp.concatenate([e[1], jnp.full((epr - E,), N, jnp.int32)]))
    edges = jnp.stack([jnp.concatenate(srcs), jnp.concatenate(dsts)])
    edges = edges.reshape(2, 3 * epr // 128, 128)

    out = _build_sc(NP, nch)(msgs.reshape(NP * 6, 64), root, edges)
    return out[:N].astype(jnp.float32)
